# baseline (speedup 1.0000x reference)
import functools

import jax
import jax.numpy as jnp
from jax.experimental import pallas as pl
from jax.experimental.pallas import tpu as pltpu

_NEG_SLOPE = 0.01
_BN_EPS = 1e-5
_POOL_SIZES = (5, 9, 13)


def _lrelu(y):
    return jnp.where(y > 0, y, _NEG_SLOPE * y)


def _fused_spp_kernel(x_ref, w0_ref, w1_ref, w2_ref, sb_ref, o_ref,
                      im2col_ref, pool_ref,
                      *, K, H, W, C1, C2, C3):
    HW = H * W
    pm = max(_POOL_SIZES) // 2
    pc = K // 2

    xb = x_ref[0].astype(jnp.bfloat16)
    y1 = jax.lax.dot_general(xb, w0_ref[...],
                             (((0,), (0,)), ((), ())),
                             preferred_element_type=jnp.float32)
    y1 = _lrelu(y1 * sb_ref[0:1, :C1] + sb_ref[1:2, :C1])

    y1b = y1.astype(jnp.bfloat16)
    w_pos = jax.lax.broadcasted_iota(jnp.int32, (HW, 1), 0) % W
    y1_neg = jnp.where(w_pos == W - 1, jnp.bfloat16(0), y1b)
    y1_pos = jnp.where(w_pos == 0, jnp.bfloat16(0), y1b)
    for t in range(K * K):
        dh, dw = t // K - pc, t % K - pc
        s = dh * W + dw
        src = y1b if dw == 0 else (y1_neg if dw < 0 else y1_pos)
        d0, d1 = max(0, -s), HW - max(0, s)
        col = slice(t * C1, (t + 1) * C1)
        if d0 > 0:
            im2col_ref[0:d0, col] = jnp.zeros((d0, C1), jnp.bfloat16)
        if d1 < HW:
            im2col_ref[d1:HW, col] = jnp.zeros((HW - d1, C1), jnp.bfloat16)
        im2col_ref[d0:d1, col] = src[d0 + s:d1 + s]
    y2 = jnp.dot(im2col_ref[...], w1_ref[...],
                 preferred_element_type=jnp.float32)
    y2 = _lrelu(y2 * sb_ref[2:3, :C2] + sb_ref[3:4, :C2]).astype(jnp.bfloat16)

    y3 = jnp.dot(y2, w2_ref[...],
                 preferred_element_type=jnp.float32)
    y3 = _lrelu(y3 * sb_ref[4:5, :C3] + sb_ref[5:6, :C3])

    o_ref[0, 0] = jnp.transpose(y3).reshape(C3, H, W)

    pool_ref[...] = jnp.full_like(pool_ref, -jnp.inf)
    pool_ref[pm:pm + H, pm * C3:(pm + W) * C3] = (
        y3.astype(jnp.bfloat16).reshape(H, W * C3))

    ext = H + 8
    r5e = pool_ref[2:2 + ext, :]
    for dd in (0, 1, 3, 4):
        r5e = jnp.maximum(r5e, pool_ref[dd:dd + ext, :])
    r5 = r5e[4:4 + H]
    r9 = jnp.maximum(r5e[2:2 + H], r5e[6:6 + H])
    r13 = jnp.maximum(jnp.maximum(r5e[0:H], r5e[4:4 + H]), r5e[8:8 + H])

    def colblocks(a, c0, n, shifts):
        res = a[:, (c0 + shifts[0]) * C3:(c0 + shifts[0] + n) * C3]
        for s in shifts[1:]:
            res = jnp.maximum(res, a[:, (c0 + s) * C3:(c0 + s + n) * C3])
        return res

    res5 = colblocks(r5, pm, W, (-2, -1, 0, 1, 2))
    m3 = colblocks(r9, pm - 3, W + 6, (-1, 0, 1))
    res9 = jnp.maximum(jnp.maximum(m3[:, 0:W * C3], m3[:, 3 * C3:(3 + W) * C3]),
                       m3[:, 6 * C3:(6 + W) * C3])
    m5 = colblocks(r13, pm - 4, W + 8, (-2, -1, 0, 1, 2))
    res13 = jnp.maximum(jnp.maximum(m5[:, 0:W * C3], m5[:, 4 * C3:(4 + W) * C3]),
                        m5[:, 8 * C3:(8 + W) * C3])

    for g, res in enumerate((res5, res9, res13)):
        rt = jnp.transpose(res.reshape(HW, C3))
        o_ref[0, g + 1] = rt.astype(o_ref.dtype).reshape(C3, H, W)


def kernel(x, w0, g0, b0, m0, v0, w1, g1, b1, m1, v1, w2, g2, b2, m2, v2):
    N, C0, H, W = x.shape
    K = w1.shape[0]
    C1, C2, C3 = w0.shape[-1], w1.shape[-1], w2.shape[-1]
    pm = max(_POOL_SIZES) // 2
    G = 1 + len(_POOL_SIZES)
    HW = H * W
    Cmax = max(C1, C2, C3)

    xr = x.reshape(N, C0, HW)
    w0r = w0.reshape(C0, C1).astype(jnp.bfloat16)
    w1r = w1.reshape(K * K * C1, C2).astype(jnp.bfloat16)
    w2r = w2.reshape(C2, C3).astype(jnp.bfloat16)

    def fold(g, b, m, v):
        s = g / jnp.sqrt(v + _BN_EPS)
        return s, b - m * s

    rows = []
    for (g, b, m, v) in ((g0, b0, m0, v0), (g1, b1, m1, v1), (g2, b2, m2, v2)):
        s, bb = fold(g, b, m, v)
        rows.append(jnp.pad(s, (0, Cmax - s.shape[0])))
        rows.append(jnp.pad(bb, (0, Cmax - bb.shape[0])))
    sb = jnp.stack(rows).astype(jnp.float32)

    kernel_fn = functools.partial(_fused_spp_kernel, K=K, H=H, W=W,
                                  C1=C1, C2=C2, C3=C3)

    out = pl.pallas_call(
        kernel_fn,
        out_shape=jax.ShapeDtypeStruct((N, G, C3, H, W), x.dtype),
        grid_spec=pltpu.PrefetchScalarGridSpec(
            num_scalar_prefetch=0,
            grid=(N,),
            in_specs=[
                pl.BlockSpec((1, C0, HW), lambda n: (n, 0, 0)),
                pl.BlockSpec((C0, C1), lambda n: (0, 0)),
                pl.BlockSpec((K * K * C1, C2), lambda n: (0, 0)),
                pl.BlockSpec((C2, C3), lambda n: (0, 0)),
                pl.BlockSpec((6, Cmax), lambda n: (0, 0)),
            ],
            out_specs=pl.BlockSpec((1, G, C3, H, W),
                                   lambda n: (n, 0, 0, 0, 0)),
            scratch_shapes=[
                pltpu.VMEM((HW, K * K * C1), jnp.bfloat16),
                pltpu.VMEM((H + 2 * pm, (W + 2 * pm) * C3), jnp.bfloat16),
            ],
        ),
        compiler_params=pltpu.CompilerParams(
            dimension_semantics=("parallel",)),
    )(xr, w0r, w1r, w2r, sb)

    return out.reshape(N, G * C3, H, W)


# speedup vs baseline: 2.8385x; 2.8385x over previous
import functools

import jax
import jax.numpy as jnp
from jax.experimental import pallas as pl
from jax.experimental.pallas import tpu as pltpu

_NEG_SLOPE = 0.01
_BN_EPS = 1e-5
_POOL_SIZES = (5, 9, 13)


def _lrelu(y):
    return jnp.where(y > 0, y, _NEG_SLOPE * y)


def _fused_spp_kernel(x_ref, w0_ref, w1_ref, w2_ref, sb_ref, o_ref,
                      im2col_ref, pool_ref,
                      *, K, H, W, C1, C2, C3):
    HW = H * W
    pm = max(_POOL_SIZES) // 2
    pc = K // 2

    xb = x_ref[0].astype(jnp.bfloat16)
    y1 = jnp.dot(xb, w0_ref[...],
                 preferred_element_type=jnp.float32)
    y1 = _lrelu(y1 * sb_ref[0:1, :C1] + sb_ref[1:2, :C1])

    y1b = y1.astype(jnp.bfloat16)
    w_pos = jax.lax.broadcasted_iota(jnp.int32, (HW, 1), 0) % W
    y1_neg = jnp.where(w_pos == W - 1, jnp.bfloat16(0), y1b)
    y1_pos = jnp.where(w_pos == 0, jnp.bfloat16(0), y1b)
    for t in range(K * K):
        dh, dw = t // K - pc, t % K - pc
        s = dh * W + dw
        src = y1b if dw == 0 else (y1_neg if dw < 0 else y1_pos)
        d0, d1 = max(0, -s), HW - max(0, s)
        col = slice(t * C1, (t + 1) * C1)
        if d0 > 0:
            im2col_ref[0:d0, col] = jnp.zeros((d0, C1), jnp.bfloat16)
        if d1 < HW:
            im2col_ref[d1:HW, col] = jnp.zeros((HW - d1, C1), jnp.bfloat16)
        im2col_ref[d0:d1, col] = src[d0 + s:d1 + s]
    y2 = jnp.dot(im2col_ref[...], w1_ref[...],
                 preferred_element_type=jnp.float32)
    y2 = _lrelu(y2 * sb_ref[2:3, :C2] + sb_ref[3:4, :C2]).astype(jnp.bfloat16)

    y3 = jnp.dot(y2, w2_ref[...],
                 preferred_element_type=jnp.float32)
    y3 = _lrelu(y3 * sb_ref[4:5, :C3] + sb_ref[5:6, :C3])

    o_ref[0, :, 0:C3] = y3

    pool_ref[...] = jnp.full_like(pool_ref, -jnp.inf)
    pool_ref[pm:pm + H, pm * C3:(pm + W) * C3] = (
        y3.astype(jnp.bfloat16).reshape(H, W * C3))

    ext = H + 8
    r5e = pool_ref[2:2 + ext, :]
    for dd in (0, 1, 3, 4):
        r5e = jnp.maximum(r5e, pool_ref[dd:dd + ext, :])
    r5 = r5e[4:4 + H]
    r9 = jnp.maximum(r5e[2:2 + H], r5e[6:6 + H])
    r13 = jnp.maximum(jnp.maximum(r5e[0:H], r5e[4:4 + H]), r5e[8:8 + H])

    def colblocks(a, c0, n, shifts):
        res = a[:, (c0 + shifts[0]) * C3:(c0 + shifts[0] + n) * C3]
        for s in shifts[1:]:
            res = jnp.maximum(res, a[:, (c0 + s) * C3:(c0 + s + n) * C3])
        return res

    res5 = colblocks(r5, pm, W, (-2, -1, 0, 1, 2))
    m3 = colblocks(r9, pm - 3, W + 6, (-1, 0, 1))
    res9 = jnp.maximum(jnp.maximum(m3[:, 0:W * C3], m3[:, 3 * C3:(3 + W) * C3]),
                       m3[:, 6 * C3:(6 + W) * C3])
    m5 = colblocks(r13, pm - 4, W + 8, (-2, -1, 0, 1, 2))
    res13 = jnp.maximum(jnp.maximum(m5[:, 0:W * C3], m5[:, 4 * C3:(4 + W) * C3]),
                        m5[:, 8 * C3:(8 + W) * C3])

    for g, res in enumerate((res5, res9, res13)):
        o_ref[0, :, (g + 1) * C3:(g + 2) * C3] = (
            res.reshape(HW, C3).astype(o_ref.dtype))


def kernel(x, w0, g0, b0, m0, v0, w1, g1, b1, m1, v1, w2, g2, b2, m2, v2):
    N, C0, H, W = x.shape
    K = w1.shape[0]
    C1, C2, C3 = w0.shape[-1], w1.shape[-1], w2.shape[-1]
    pm = max(_POOL_SIZES) // 2
    G = 1 + len(_POOL_SIZES)
    HW = H * W
    Cmax = max(C1, C2, C3)

    xr = jnp.transpose(x, (0, 2, 3, 1)).reshape(N, HW, C0)
    w0r = w0.reshape(C0, C1).astype(jnp.bfloat16)
    w1r = w1.reshape(K * K * C1, C2).astype(jnp.bfloat16)
    w2r = w2.reshape(C2, C3).astype(jnp.bfloat16)

    def fold(g, b, m, v):
        s = g / jnp.sqrt(v + _BN_EPS)
        return s, b - m * s

    rows = []
    for (g, b, m, v) in ((g0, b0, m0, v0), (g1, b1, m1, v1), (g2, b2, m2, v2)):
        s, bb = fold(g, b, m, v)
        rows.append(jnp.pad(s, (0, Cmax - s.shape[0])))
        rows.append(jnp.pad(bb, (0, Cmax - bb.shape[0])))
    sb = jnp.stack(rows).astype(jnp.float32)

    kernel_fn = functools.partial(_fused_spp_kernel, K=K, H=H, W=W,
                                  C1=C1, C2=C2, C3=C3)

    out = pl.pallas_call(
        kernel_fn,
        out_shape=jax.ShapeDtypeStruct((N, HW, G * C3), x.dtype),
        grid_spec=pltpu.PrefetchScalarGridSpec(
            num_scalar_prefetch=0,
            grid=(N,),
            in_specs=[
                pl.BlockSpec((1, HW, C0), lambda n: (n, 0, 0)),
                pl.BlockSpec((C0, C1), lambda n: (0, 0)),
                pl.BlockSpec((K * K * C1, C2), lambda n: (0, 0)),
                pl.BlockSpec((C2, C3), lambda n: (0, 0)),
                pl.BlockSpec((6, Cmax), lambda n: (0, 0)),
            ],
            out_specs=pl.BlockSpec((1, HW, G * C3), lambda n: (n, 0, 0)),
            scratch_shapes=[
                pltpu.VMEM((HW, K * K * C1), jnp.bfloat16),
                pltpu.VMEM((H + 2 * pm, (W + 2 * pm) * C3), jnp.bfloat16),
            ],
        ),
        compiler_params=pltpu.CompilerParams(
            dimension_semantics=("parallel",)),
    )(xr, w0r, w1r, w2r, sb)

    return jnp.transpose(out.reshape(N, H, W, G * C3), (0, 3, 1, 2))


# speedup vs baseline: 2.8708x; 1.0114x over previous
import functools

import jax
import jax.numpy as jnp
from jax.experimental import pallas as pl
from jax.experimental.pallas import tpu as pltpu

_NEG_SLOPE = 0.01
_BN_EPS = 1e-5
_POOL_SIZES = (5, 9, 13)


def _lrelu(y):
    return jnp.where(y > 0, y, _NEG_SLOPE * y)


def _fused_spp_kernel(x_ref, w0_ref, w1_ref, w2_ref, sb_ref, o_ref,
                      im2col_ref, pool_ref,
                      *, K, H, W, C1, C2, C3):
    HW = H * W
    pm = max(_POOL_SIZES) // 2
    pc = K // 2

    xb = x_ref[0].astype(jnp.bfloat16)
    y1 = jnp.dot(xb, w0_ref[...],
                 preferred_element_type=jnp.float32)
    sbb = sb_ref[...].astype(jnp.bfloat16)
    y1b = _lrelu(y1.astype(jnp.bfloat16) * sbb[0:1, :C1] + sbb[1:2, :C1])

    w_pos = jax.lax.broadcasted_iota(jnp.int32, (HW, 1), 0) % W
    y1_neg = jnp.where(w_pos == W - 1, jnp.bfloat16(0), y1b)
    y1_pos = jnp.where(w_pos == 0, jnp.bfloat16(0), y1b)
    for t in range(K * K):
        dh, dw = t // K - pc, t % K - pc
        s = dh * W + dw
        src = y1b if dw == 0 else (y1_neg if dw < 0 else y1_pos)
        d0, d1 = max(0, -s), HW - max(0, s)
        col = slice(t * C1, (t + 1) * C1)
        if d0 > 0:
            im2col_ref[0:d0, col] = jnp.zeros((d0, C1), jnp.bfloat16)
        if d1 < HW:
            im2col_ref[d1:HW, col] = jnp.zeros((HW - d1, C1), jnp.bfloat16)
        im2col_ref[d0:d1, col] = src[d0 + s:d1 + s]
    y2 = jnp.dot(im2col_ref[...], w1_ref[...],
                 preferred_element_type=jnp.float32)
    y2 = _lrelu(y2.astype(jnp.bfloat16) * sbb[2:3, :C2] + sbb[3:4, :C2])

    y3 = jnp.dot(y2, w2_ref[...],
                 preferred_element_type=jnp.float32)
    y3 = _lrelu(y3 * sb_ref[4:5, :C3] + sb_ref[5:6, :C3])

    o_ref[0, :, 0:C3] = y3

    pool_ref[...] = jnp.full_like(pool_ref, -jnp.inf)
    pool_ref[pm:pm + H, pm * C3:(pm + W) * C3] = (
        y3.astype(jnp.bfloat16).reshape(H, W * C3))

    ext = H + 8
    r5e = pool_ref[2:2 + ext, :]
    for dd in (0, 1, 3, 4):
        r5e = jnp.maximum(r5e, pool_ref[dd:dd + ext, :])
    r5 = r5e[4:4 + H]
    r9 = jnp.maximum(r5e[2:2 + H], r5e[6:6 + H])
    r13 = jnp.maximum(jnp.maximum(r5e[0:H], r5e[4:4 + H]), r5e[8:8 + H])

    def colblocks(a, c0, n, shifts):
        res = a[:, (c0 + shifts[0]) * C3:(c0 + shifts[0] + n) * C3]
        for s in shifts[1:]:
            res = jnp.maximum(res, a[:, (c0 + s) * C3:(c0 + s + n) * C3])
        return res

    res5 = colblocks(r5, pm, W, (-2, -1, 0, 1, 2))
    m3 = colblocks(r9, pm - 3, W + 6, (-1, 0, 1))
    res9 = jnp.maximum(jnp.maximum(m3[:, 0:W * C3], m3[:, 3 * C3:(3 + W) * C3]),
                       m3[:, 6 * C3:(6 + W) * C3])
    m5 = colblocks(r13, pm - 4, W + 8, (-2, -1, 0, 1, 2))
    res13 = jnp.maximum(jnp.maximum(m5[:, 0:W * C3], m5[:, 4 * C3:(4 + W) * C3]),
                        m5[:, 8 * C3:(8 + W) * C3])

    for g, res in enumerate((res5, res9, res13)):
        o_ref[0, :, (g + 1) * C3:(g + 2) * C3] = (
            res.reshape(HW, C3).astype(o_ref.dtype))


def kernel(x, w0, g0, b0, m0, v0, w1, g1, b1, m1, v1, w2, g2, b2, m2, v2):
    N, C0, H, W = x.shape
    K = w1.shape[0]
    C1, C2, C3 = w0.shape[-1], w1.shape[-1], w2.shape[-1]
    pm = max(_POOL_SIZES) // 2
    G = 1 + len(_POOL_SIZES)
    HW = H * W
    Cmax = max(C1, C2, C3)

    xr = jnp.transpose(x, (0, 2, 3, 1)).reshape(N, HW, C0)
    w0r = w0.reshape(C0, C1).astype(jnp.bfloat16)
    w1r = w1.reshape(K * K * C1, C2).astype(jnp.bfloat16)
    w2r = w2.reshape(C2, C3).astype(jnp.bfloat16)

    def fold(g, b, m, v):
        s = g / jnp.sqrt(v + _BN_EPS)
        return s, b - m * s

    rows = []
    for (g, b, m, v) in ((g0, b0, m0, v0), (g1, b1, m1, v1), (g2, b2, m2, v2)):
        s, bb = fold(g, b, m, v)
        rows.append(jnp.pad(s, (0, Cmax - s.shape[0])))
        rows.append(jnp.pad(bb, (0, Cmax - bb.shape[0])))
    sb = jnp.stack(rows).astype(jnp.float32)

    kernel_fn = functools.partial(_fused_spp_kernel, K=K, H=H, W=W,
                                  C1=C1, C2=C2, C3=C3)

    out = pl.pallas_call(
        kernel_fn,
        out_shape=jax.ShapeDtypeStruct((N, HW, G * C3), x.dtype),
        grid_spec=pltpu.PrefetchScalarGridSpec(
            num_scalar_prefetch=0,
            grid=(N,),
            in_specs=[
                pl.BlockSpec((1, HW, C0), lambda n: (n, 0, 0)),
                pl.BlockSpec((C0, C1), lambda n: (0, 0)),
                pl.BlockSpec((K * K * C1, C2), lambda n: (0, 0)),
                pl.BlockSpec((C2, C3), lambda n: (0, 0)),
                pl.BlockSpec((6, Cmax), lambda n: (0, 0)),
            ],
            out_specs=pl.BlockSpec((1, HW, G * C3), lambda n: (n, 0, 0)),
            scratch_shapes=[
                pltpu.VMEM((HW, K * K * C1), jnp.bfloat16),
                pltpu.VMEM((H + 2 * pm, (W + 2 * pm) * C3), jnp.bfloat16),
            ],
        ),
        compiler_params=pltpu.CompilerParams(
            dimension_semantics=("parallel",)),
    )(xr, w0r, w1r, w2r, sb)

    return jnp.transpose(out.reshape(N, H, W, G * C3), (0, 3, 1, 2))
